# baseline (speedup 1.0000x reference)
import jax
import jax.numpy as jnp
from jax.experimental import pallas as pl
from jax.experimental.pallas import tpu as pltpu

HIDDEN = 256
EPS = 1e-6
TILE = 256


def _block_kernel(x_full_ref, x_tile_ref, p_ref, w_ref, b_ref,
                  g1_ref, b1_ref, g2_ref, b2_ref, o_ref, h_ref):
    t = pl.program_id(1)

    @pl.when(t == 0)
    def _compute_h():
        x = x_full_ref[0]
        mu = jnp.mean(x, axis=-1, keepdims=True)
        xc = x - mu
        var = jnp.mean(xc * xc, axis=-1, keepdims=True)
        h_ref[...] = g1_ref[0] * xc / jnp.sqrt(var + EPS) + b1_ref[0]

    p = p_ref[0]
    agg = jnp.dot(p, h_ref[...], preferred_element_type=jnp.float32)
    conv = jnp.maximum(
        jnp.dot(agg, w_ref[...], preferred_element_type=jnp.float32)
        + b_ref[0], 0.0)

    y = x_tile_ref[0] + conv
    mu2 = jnp.mean(y, axis=-1, keepdims=True)
    yc = y - mu2
    var2 = jnp.mean(yc * yc, axis=-1, keepdims=True)
    o_ref[0] = g2_ref[0] * yc / jnp.sqrt(var2 + EPS) + b2_ref[0]


def kernel(x, mask, inputP, W, b, ln1_g, ln1_b, ln2_g, ln2_b):
    del mask
    B, N, H = x.shape
    T = N // TILE

    vec = lambda v: v.reshape(1, H)
    return pl.pallas_call(
        _block_kernel,
        grid=(B, T),
        in_specs=[
            pl.BlockSpec((1, N, H), lambda i, t: (i, 0, 0)),
            pl.BlockSpec((1, TILE, H), lambda i, t: (i, t, 0)),
            pl.BlockSpec((1, TILE, N), lambda i, t: (i, t, 0)),
            pl.BlockSpec((H, H), lambda i, t: (0, 0)),
            pl.BlockSpec((1, H), lambda i, t: (0, 0)),
            pl.BlockSpec((1, H), lambda i, t: (0, 0)),
            pl.BlockSpec((1, H), lambda i, t: (0, 0)),
            pl.BlockSpec((1, H), lambda i, t: (0, 0)),
            pl.BlockSpec((1, H), lambda i, t: (0, 0)),
        ],
        out_specs=pl.BlockSpec((1, TILE, H), lambda i, t: (i, t, 0)),
        out_shape=jax.ShapeDtypeStruct((B, N, H), x.dtype),
        scratch_shapes=[pltpu.VMEM((N, H), jnp.float32)],
        compiler_params=pltpu.CompilerParams(
            dimension_semantics=("arbitrary", "arbitrary")),
    )(x, x, inputP, W, vec(b), vec(ln1_g), vec(ln1_b), vec(ln2_g), vec(ln2_b))
